# Initial kernel scaffold; baseline (speedup 1.0000x reference)
import jax
import jax.numpy as jnp
from jax.experimental import pallas as pl


def kernel(x, edge_index, Ws, bs):
    raise NotImplementedError("write your pallas kernel here")


# speedup vs baseline: 7.9615x; 7.9615x over previous
import functools

import jax
import jax.numpy as jnp
from jax import lax
from jax.experimental import pallas as pl
from jax.experimental.pallas import tpu as pltpu
from jax.experimental.pallas import tpu_sc as plsc

N = 10000
NPAD = 10240
E = 320000
D = 128
TILES_PER_CORE = 16
NUM_CORES = 2
NUM_TILES = TILES_PER_CORE * NUM_CORES
STRIPE = NPAD // TILES_PER_CORE
EDGES_PER_TILE = E // NUM_TILES
CHUNK = 125
NCHUNK = EDGES_PER_TILE // CHUNK
BLK = 1000
NBLK = N // BLK


@functools.lru_cache(None)
def _scatter_call():
    mesh = plsc.VectorSubcoreMesh(core_axis_name="c", subcore_axis_name="s")

    def body(u_hbm, row_hbm, col_hbm, zrows_hbm, out0_hbm, out1_hbm,
             row_v, col_v, buf_v, acc_sh):
        c = lax.axis_index("c")
        s = lax.axis_index("s")
        wid = c * TILES_PER_CORE + s
        pltpu.sync_copy(zrows_hbm, acc_sh.at[pl.ds(s * STRIPE, STRIPE)])
        pltpu.sync_copy(row_hbm.at[wid], row_v)
        pltpu.sync_copy(col_hbm.at[wid], col_v)
        plsc.subcore_barrier()

        def step(i, carry):
            pltpu.sync_copy(u_hbm.at[row_v.at[i]], buf_v)
            pltpu.sync_copy(buf_v, acc_sh.at[col_v.at[i]], add=True)
            return carry

        lax.fori_loop(0, NCHUNK, step, 0)
        plsc.subcore_barrier()

        @pl.when(c == 0)
        def _():
            pltpu.sync_copy(acc_sh.at[pl.ds(s * STRIPE, STRIPE)],
                            out0_hbm.at[pl.ds(s * STRIPE, STRIPE)])

        @pl.when(c == 1)
        def _():
            pltpu.sync_copy(acc_sh.at[pl.ds(s * STRIPE, STRIPE)],
                            out1_hbm.at[pl.ds(s * STRIPE, STRIPE)])

    return pl.kernel(
        body,
        out_type=[jax.ShapeDtypeStruct((NPAD, D), jnp.float32),
                  jax.ShapeDtypeStruct((NPAD, D), jnp.float32)],
        mesh=mesh,
        scratch_types=[
            pltpu.VMEM((NCHUNK, CHUNK), jnp.int32),
            pltpu.VMEM((NCHUNK, CHUNK), jnp.int32),
            pltpu.VMEM((CHUNK, D), jnp.float32),
            pltpu.VMEM_SHARED((NPAD, D), jnp.float32),
        ],
    )


def _row_block(i):
    return (i, 0)


@functools.lru_cache(None)
def _dis_call():
    def body(dqa_ref, dqb_ref, x_ref, dis_ref, u_ref):
        deg = dqa_ref[:, 0:1] + dqb_ref[:, 0:1]
        d = jnp.where(deg > 0.0,
                      1.0 / jnp.sqrt(jnp.where(deg > 0.0, deg, 1.0)), 0.0)
        dis = jnp.broadcast_to(d, (BLK, D))
        dis_ref[...] = dis
        u_ref[...] = dis * x_ref[...]

    return pl.pallas_call(
        body,
        grid=(NBLK,),
        in_specs=[pl.BlockSpec((BLK, D), _row_block)] * 3,
        out_specs=[pl.BlockSpec((BLK, D), _row_block)] * 2,
        out_shape=[jax.ShapeDtypeStruct((N, D), jnp.float32)] * 2,
    )


@functools.lru_cache(None)
def _stage_call(first):
    def body(*refs):
        if first:
            qa_ref, qb_ref, dis_ref, t_ref, u_ref = refs
        else:
            qa_ref, qb_ref, dis_ref, tp_ref, t_ref, u_ref = refs
        dis = dis_ref[...]
        ssum = qa_ref[...] + qb_ref[...]
        if first:
            t = -dis * ssum
        else:
            t = -2.0 * dis * ssum - tp_ref[...]
        t_ref[...] = t
        u_ref[...] = dis * t

    n_in = 3 if first else 4
    return pl.pallas_call(
        body,
        grid=(NBLK,),
        in_specs=[pl.BlockSpec((BLK, D), _row_block)] * n_in,
        out_specs=[pl.BlockSpec((BLK, D), _row_block)] * 2,
        out_shape=[jax.ShapeDtypeStruct((N, D), jnp.float32)] * 2,
    )


@functools.lru_cache(None)
def _matmul_call(last):
    def body(*refs):
        t_refs = refs[0:5]
        if last:
            w_ref, b_ref, h_ref = refs[5:]
        else:
            w_ref, b_ref, dis_ref, h_ref, u_ref = refs[5:]
        acc = jnp.zeros((BLK, D), jnp.float32)
        for k in range(5):
            acc = acc + jnp.dot(t_refs[k][...],
                                w_ref[k * D:(k + 1) * D, :],
                                preferred_element_type=jnp.float32)
        acc = acc + b_ref[...]
        if last:
            h_ref[...] = acc
        else:
            acc = jnp.maximum(acc, 0.0)
            h_ref[...] = acc
            u_ref[...] = dis_ref[...] * acc

    n_extra = 2 if last else 3
    n_out = 1 if last else 2
    in_specs = [pl.BlockSpec((BLK, D), _row_block) for _ in range(5)]
    in_specs.append(pl.BlockSpec((5 * D, D), lambda i: (0, 0)))
    in_specs.append(pl.BlockSpec((1, D), lambda i: (0, 0)))
    if not last:
        in_specs.append(pl.BlockSpec((BLK, D), _row_block))
    return pl.pallas_call(
        body,
        grid=(NBLK,),
        in_specs=in_specs,
        out_specs=[pl.BlockSpec((BLK, D), _row_block)] * n_out,
        out_shape=[jax.ShapeDtypeStruct((N, D), jnp.float32)] * n_out,
    )


def kernel(x, edge_index, Ws, bs):
    row3 = edge_index[0].reshape(NUM_TILES, NCHUNK, CHUNK)
    col3 = edge_index[1].reshape(NUM_TILES, NCHUNK, CHUNK)
    zrows = jnp.zeros((STRIPE, D), jnp.float32)

    n_layers = len(Ws)
    Wp, bp = [], []
    for W, b in zip(Ws, bs):
        din, dout = W.shape[1], W.shape[2]
        Wp.append(jnp.pad(W, ((0, 0), (0, D - din), (0, D - dout)))
                  .reshape(5 * D, D))
        bp.append(jnp.pad(b, (0, D - dout)).reshape(1, D))

    ones_u = jnp.ones((N, D), jnp.float32)
    deg0, deg1 = _scatter_call()(ones_u, row3, col3, zrows)
    dis, u = _dis_call()(deg0, deg1, x)

    h = x
    for l in range(n_layers):
        ts = [h]
        for k in range(1, 5):
            q0, q1 = _scatter_call()(u, row3, col3, zrows)
            if k == 1:
                t, u = _stage_call(True)(q0, q1, dis)
            else:
                t, u = _stage_call(False)(q0, q1, dis, ts[k - 2])
            ts.append(t)
        if l == n_layers - 1:
            hfull = _matmul_call(True)(*ts, Wp[l], bp[l])
            return hfull[0][:, :Ws[l].shape[2]]
        h, u = _matmul_call(False)(*ts, Wp[l], bp[l], dis)
    return h


# speedup vs baseline: 8.7889x; 1.1039x over previous
import functools

import jax
import jax.numpy as jnp
from jax import lax
from jax.experimental import pallas as pl
from jax.experimental.pallas import tpu as pltpu
from jax.experimental.pallas import tpu_sc as plsc

N = 10000
NPAD = 10112
E = 320000
D = 128
TILES_PER_CORE = 16
NUM_CORES = 2
NUM_TILES = TILES_PER_CORE * NUM_CORES
STRIPE = NPAD // TILES_PER_CORE
EDGES_PER_TILE = E // NUM_TILES
CHUNK = 80
NCHUNK = EDGES_PER_TILE // CHUNK
BLK = 1000
NBLK = N // BLK


@functools.lru_cache(None)
def _scatter_call():
    mesh = plsc.VectorSubcoreMesh(core_axis_name="c", subcore_axis_name="s")

    def body(u_hbm, row_hbm, col_hbm, zrows_hbm, out0_hbm, out1_hbm,
             row_v, col_v, b0, b1, g0, g1, s0, s1, acc_sh):
        bufs = (b0, b1)
        gsems = (g0, g1)
        ssems = (s0, s1)
        c = lax.axis_index("c")
        s = lax.axis_index("s")
        wid = c * TILES_PER_CORE + s
        pltpu.sync_copy(zrows_hbm, acc_sh.at[pl.ds(s * STRIPE, STRIPE)])
        pltpu.sync_copy(row_hbm.at[wid], row_v)
        pltpu.sync_copy(col_hbm.at[wid], col_v)
        pltpu.async_copy(u_hbm.at[row_v.at[pl.ds(0, CHUNK)]], bufs[0], gsems[0])
        plsc.subcore_barrier()

        def outer(j, carry):
            for b in range(2):
                i = 2 * j + b
                o = 1 - b
                pltpu.make_async_copy(
                    u_hbm.at[row_v.at[pl.ds(i * CHUNK, CHUNK)]],
                    bufs[b], gsems[b]).wait()
                pltpu.async_copy(
                    bufs[b], acc_sh.at[col_v.at[i]], ssems[b], add=True)
                if b == 0:
                    @pl.when(j > 0)
                    def _():
                        pltpu.make_async_copy(
                            bufs[o], acc_sh.at[col_v.at[i - 1]],
                            ssems[o]).wait()
                else:
                    pltpu.make_async_copy(
                        bufs[o], acc_sh.at[col_v.at[i - 1]], ssems[o]).wait()
                pltpu.async_copy(
                    u_hbm.at[row_v.at[pl.ds((i + 1) * CHUNK, CHUNK)]],
                    bufs[o], gsems[o])
            return carry

        lax.fori_loop(0, (NCHUNK - 1) // 2, outer, 0)
        pltpu.make_async_copy(
            u_hbm.at[row_v.at[pl.ds((NCHUNK - 1) * CHUNK, CHUNK)]],
            bufs[0], gsems[0]).wait()
        pltpu.async_copy(
            bufs[0], acc_sh.at[col_v.at[NCHUNK - 1]], ssems[0], add=True)
        pltpu.make_async_copy(
            bufs[1], acc_sh.at[col_v.at[NCHUNK - 2]], ssems[1]).wait()
        pltpu.make_async_copy(
            bufs[0], acc_sh.at[col_v.at[NCHUNK - 1]], ssems[0]).wait()
        plsc.subcore_barrier()

        @pl.when(c == 0)
        def _():
            pltpu.sync_copy(acc_sh.at[pl.ds(s * STRIPE, STRIPE)],
                            out0_hbm.at[pl.ds(s * STRIPE, STRIPE)])

        @pl.when(c == 1)
        def _():
            pltpu.sync_copy(acc_sh.at[pl.ds(s * STRIPE, STRIPE)],
                            out1_hbm.at[pl.ds(s * STRIPE, STRIPE)])

    return pl.kernel(
        body,
        out_type=[jax.ShapeDtypeStruct((NPAD, D), jnp.float32),
                  jax.ShapeDtypeStruct((NPAD, D), jnp.float32)],
        mesh=mesh,
        scratch_types=(
            [pltpu.VMEM((EDGES_PER_TILE,), jnp.int32),
               pltpu.VMEM((NCHUNK, CHUNK), jnp.int32)]
            + [pltpu.VMEM((CHUNK, D), jnp.float32)] * 2
            + [pltpu.SemaphoreType.DMA] * 4
            + [pltpu.VMEM_SHARED((NPAD, D), jnp.float32)]
        ),
    )


def _row_block(i):
    return (i, 0)


@functools.lru_cache(None)
def _dis_call():
    def body(dqa_ref, dqb_ref, x_ref, dis_ref, u_ref):
        deg = dqa_ref[:, 0:1] + dqb_ref[:, 0:1]
        d = jnp.where(deg > 0.0,
                      1.0 / jnp.sqrt(jnp.where(deg > 0.0, deg, 1.0)), 0.0)
        dis = jnp.broadcast_to(d, (BLK, D))
        dis_ref[...] = dis
        u_ref[...] = dis * x_ref[...]

    return pl.pallas_call(
        body,
        grid=(NBLK,),
        in_specs=[pl.BlockSpec((BLK, D), _row_block)] * 3,
        out_specs=[pl.BlockSpec((BLK, D), _row_block)] * 2,
        out_shape=[jax.ShapeDtypeStruct((N, D), jnp.float32)] * 2,
    )


@functools.lru_cache(None)
def _stage_call(first):
    def body(*refs):
        if first:
            qa_ref, qb_ref, dis_ref, t_ref, u_ref = refs
        else:
            qa_ref, qb_ref, dis_ref, tp_ref, t_ref, u_ref = refs
        dis = dis_ref[...]
        ssum = qa_ref[...] + qb_ref[...]
        if first:
            t = -dis * ssum
        else:
            t = -2.0 * dis * ssum - tp_ref[...]
        t_ref[...] = t
        u_ref[...] = dis * t

    n_in = 3 if first else 4
    return pl.pallas_call(
        body,
        grid=(NBLK,),
        in_specs=[pl.BlockSpec((BLK, D), _row_block)] * n_in,
        out_specs=[pl.BlockSpec((BLK, D), _row_block)] * 2,
        out_shape=[jax.ShapeDtypeStruct((N, D), jnp.float32)] * 2,
    )


@functools.lru_cache(None)
def _matmul_call(last):
    def body(*refs):
        t_refs = refs[0:5]
        if last:
            w_ref, b_ref, h_ref = refs[5:]
        else:
            w_ref, b_ref, dis_ref, h_ref, u_ref = refs[5:]
        acc = jnp.zeros((BLK, D), jnp.float32)
        for k in range(5):
            acc = acc + jnp.dot(t_refs[k][...],
                                w_ref[k * D:(k + 1) * D, :],
                                preferred_element_type=jnp.float32)
        acc = acc + b_ref[...]
        if last:
            h_ref[...] = acc
        else:
            acc = jnp.maximum(acc, 0.0)
            h_ref[...] = acc
            u_ref[...] = dis_ref[...] * acc

    n_extra = 2 if last else 3
    n_out = 1 if last else 2
    in_specs = [pl.BlockSpec((BLK, D), _row_block) for _ in range(5)]
    in_specs.append(pl.BlockSpec((5 * D, D), lambda i: (0, 0)))
    in_specs.append(pl.BlockSpec((1, D), lambda i: (0, 0)))
    if not last:
        in_specs.append(pl.BlockSpec((BLK, D), _row_block))
    return pl.pallas_call(
        body,
        grid=(NBLK,),
        in_specs=in_specs,
        out_specs=[pl.BlockSpec((BLK, D), _row_block)] * n_out,
        out_shape=[jax.ShapeDtypeStruct((N, D), jnp.float32)] * n_out,
    )


def kernel(x, edge_index, Ws, bs):
    row3 = edge_index[0].reshape(NUM_TILES, EDGES_PER_TILE)
    col3 = edge_index[1].reshape(NUM_TILES, NCHUNK, CHUNK)
    zrows = jnp.zeros((STRIPE, D), jnp.float32)

    n_layers = len(Ws)
    Wp, bp = [], []
    for W, b in zip(Ws, bs):
        din, dout = W.shape[1], W.shape[2]
        Wp.append(jnp.pad(W, ((0, 0), (0, D - din), (0, D - dout)))
                  .reshape(5 * D, D))
        bp.append(jnp.pad(b, (0, D - dout)).reshape(1, D))

    ones_u = jnp.ones((N, D), jnp.float32)
    deg0, deg1 = _scatter_call()(ones_u, row3, col3, zrows)
    dis, u = _dis_call()(deg0, deg1, x)

    h = x
    for l in range(n_layers):
        ts = [h]
        for k in range(1, 5):
            q0, q1 = _scatter_call()(u, row3, col3, zrows)
            if k == 1:
                t, u = _stage_call(True)(q0, q1, dis)
            else:
                t, u = _stage_call(False)(q0, q1, dis, ts[k - 2])
            ts.append(t)
        if l == n_layers - 1:
            hfull = _matmul_call(True)(*ts, Wp[l], bp[l])
            return hfull[0][:, :Ws[l].shape[2]]
        h, u = _matmul_call(False)(*ts, Wp[l], bp[l], dis)
    return h
